# baseline (speedup 1.0000x reference)
import functools
import math

import jax
import jax.numpy as jnp
from jax import lax
from jax.experimental import pallas as pl
from jax.experimental.pallas import tpu as pltpu
from jax.experimental.pallas import tpu_sc as plsc

VOCAB = 32000
PAD = 0
SMOOTH = 0.1
CONF = 1.0 - SMOOTH
EPS = SMOOTH / (VOCAB - 1)
KCONST = (VOCAB - 1) * EPS * math.log(EPS) + CONF * math.log(CONF)

ROWS = 2048
LANES = 128
XROWS = ROWS * (VOCAB // LANES)

NC, NS, L = 2, 16, 16
NW = NC * NS
BPW = ROWS // NW
CHUNKS = BPW // L

R = 64
NB = ROWS // R


@functools.partial(
    pl.kernel,
    out_type=jax.ShapeDtypeStruct((NW * L,), jnp.float32),
    mesh=plsc.VectorSubcoreMesh(core_axis_name="c", subcore_axis_name="s"),
    scratch_types=[
        pltpu.VMEM((BPW,), jnp.int32),
        pltpu.VMEM((BPW,), jnp.int32),
        pltpu.VMEM((BPW,), jnp.float32),
        pltpu.VMEM((L,), jnp.float32),
        pltpu.SemaphoreType.DMA,
    ],
)
def _sc_gather(xflat_hbm, tgt_hbm, out_hbm, tgt_v, idx_v, vals_v, acc_v, sem):
    wid = lax.axis_index("s") * NC + lax.axis_index("c")
    base = wid * BPW
    pltpu.sync_copy(tgt_hbm.at[pl.ds(base, BPW)], tgt_v)
    for k in range(CHUNKS):
        t16 = tgt_v[pl.ds(k * L, L)]
        row16 = (lax.iota(jnp.int32, L) + (base + k * L)) * VOCAB + t16
        idx_v[pl.ds(k * L, L)] = row16
    pltpu.async_copy(xflat_hbm.at[idx_v], vals_v, sem).wait()
    acc = jnp.zeros((L,), jnp.float32)
    for k in range(CHUNKS):
        t16 = tgt_v[pl.ds(k * L, L)]
        v16 = vals_v[pl.ds(k * L, L)]
        acc = acc + jnp.where(t16 != PAD, v16, 0.0)
    acc_v[...] = acc
    pltpu.sync_copy(acc_v, out_hbm.at[pl.ds(wid * L, L)])


def _tc_body(x_ref, t_ref, s_ref, n_ref, acc_s, acc_n):
    i = pl.program_id(0)

    @pl.when(i == 0)
    def _init():
        acc_s[0] = 0.0
        acc_n[0] = 0.0

    xb = x_ref[...]
    tb = t_ref[0, 0, :]
    maskf = (tb != PAD).astype(jnp.float32)[:, None]
    acc_s[0] += jnp.sum(xb * maskf)
    acc_n[0] += jnp.sum(maskf)

    @pl.when(i == NB - 1)
    def _fin():
        s_ref[0, 0] = acc_s[0]
        n_ref[0, 0] = acc_n[0]


def _combine_body(g_ref, s_ref, n_ref, out_ref):
    tok = n_ref[0, 0]
    g = jnp.sum(g_ref[...])
    num = KCONST * tok - EPS * s_ref[0, 0] + (EPS - CONF) * g
    out_ref[0, 0] = num / tok


@jax.jit
def _tc_stream(x, t3):
    return pl.pallas_call(
        _tc_body,
        grid=(NB,),
        in_specs=[
            pl.BlockSpec((R, VOCAB), lambda i: (i, 0)),
            pl.BlockSpec((1, 1, R), lambda i: (i, 0, 0)),
        ],
        out_specs=[
            pl.BlockSpec(memory_space=pltpu.SMEM),
            pl.BlockSpec(memory_space=pltpu.SMEM),
        ],
        out_shape=[
            jax.ShapeDtypeStruct((1, 1), jnp.float32),
            jax.ShapeDtypeStruct((1, 1), jnp.float32),
        ],
        scratch_shapes=[
            pltpu.SMEM((1,), jnp.float32),
            pltpu.SMEM((1,), jnp.float32),
        ],
    )(x, t3)


@jax.jit
def _combine(g2, s, n):
    return pl.pallas_call(
        _combine_body,
        in_specs=[
            pl.BlockSpec((1, NW * L), lambda: (0, 0)),
            pl.BlockSpec(memory_space=pltpu.SMEM),
            pl.BlockSpec(memory_space=pltpu.SMEM),
        ],
        out_specs=pl.BlockSpec(memory_space=pltpu.SMEM),
        out_shape=jax.ShapeDtypeStruct((1, 1), jnp.float32),
    )(g2, s, n)


def _tcsg_body(x_ref, t_ref, s_ref, n_ref, g_ref, acc_s, acc_n, acc_g):
    i = pl.program_id(0)

    @pl.when(i == 0)
    def _init():
        acc_s[0] = 0.0
        acc_n[0] = 0.0
        acc_g[0] = 0.0

    xb = x_ref[...]
    tb = t_ref[0, 0, :]
    maskf = (tb != PAD).astype(jnp.float32)[:, None]
    xm = xb * maskf
    cols = lax.broadcasted_iota(jnp.int32, (R, VOCAB), 1)
    sel = (cols == tb[:, None]).astype(jnp.float32)
    acc_s[0] += jnp.sum(xm)
    acc_n[0] += jnp.sum(maskf)
    acc_g[0] += jnp.sum(xm * sel)

    @pl.when(i == NB - 1)
    def _fin():
        s_ref[0, 0] = acc_s[0]
        n_ref[0, 0] = acc_n[0]
        g_ref[0, 0] = acc_g[0]


@jax.jit
def _tc_stream_g(x, t3):
    return pl.pallas_call(
        _tcsg_body,
        grid=(NB,),
        in_specs=[
            pl.BlockSpec((R, VOCAB), lambda i: (i, 0)),
            pl.BlockSpec((1, 1, R), lambda i: (i, 0, 0)),
        ],
        out_specs=[
            pl.BlockSpec(memory_space=pltpu.SMEM),
            pl.BlockSpec(memory_space=pltpu.SMEM),
            pl.BlockSpec(memory_space=pltpu.SMEM),
        ],
        out_shape=[
            jax.ShapeDtypeStruct((1, 1), jnp.float32),
            jax.ShapeDtypeStruct((1, 1), jnp.float32),
            jax.ShapeDtypeStruct((1, 1), jnp.float32),
        ],
        scratch_shapes=[
            pltpu.SMEM((1,), jnp.float32),
            pltpu.SMEM((1,), jnp.float32),
            pltpu.SMEM((1,), jnp.float32),
        ],
    )(x, t3)


def kernel(x, target):
    t = target.astype(jnp.int32)
    s, n = _tc_stream(x, t.reshape(NB, 1, R))
    g = _sc_gather(x.reshape(ROWS * VOCAB), t)
    return _combine(g.reshape(1, NW * L), s, n)[0, 0]


# speedup vs baseline: 1.0064x; 1.0064x over previous
import functools
import math

import jax
import jax.numpy as jnp
from jax import lax
from jax.experimental import pallas as pl
from jax.experimental.pallas import tpu as pltpu
from jax.experimental.pallas import tpu_sc as plsc

VOCAB = 32000
PAD = 0
SMOOTH = 0.1
CONF = 1.0 - SMOOTH
EPS = SMOOTH / (VOCAB - 1)
KCONST = (VOCAB - 1) * EPS * math.log(EPS) + CONF * math.log(CONF)

ROWS = 2048
LANES = 128
XROWS = ROWS * (VOCAB // LANES)

NC, NS, L = 1, 16, 16
NW = NC * NS
BPW = ROWS // NW
CHUNKS = BPW // L

R = 64
NB = ROWS // R


@functools.partial(
    pl.kernel,
    out_type=jax.ShapeDtypeStruct((NW * L,), jnp.float32),
    mesh=plsc.VectorSubcoreMesh(
        core_axis_name="c", subcore_axis_name="s", num_cores=NC
    ),
    scratch_types=[
        pltpu.VMEM((BPW,), jnp.int32),
        pltpu.VMEM((BPW,), jnp.int32),
        pltpu.VMEM((BPW,), jnp.float32),
        pltpu.VMEM((L,), jnp.float32),
        pltpu.SemaphoreType.DMA,
    ],
)
def _sc_gather(xflat_hbm, tgt_hbm, out_hbm, tgt_v, idx_v, vals_v, acc_v, sem):
    wid = lax.axis_index("s") * NC + lax.axis_index("c")
    base = wid * BPW
    pltpu.sync_copy(tgt_hbm.at[pl.ds(base, BPW)], tgt_v)
    for k in range(CHUNKS):
        t16 = tgt_v[pl.ds(k * L, L)]
        row16 = (lax.iota(jnp.int32, L) + (base + k * L)) * VOCAB + t16
        idx_v[pl.ds(k * L, L)] = row16
    pltpu.async_copy(xflat_hbm.at[idx_v], vals_v, sem).wait()
    acc = jnp.zeros((L,), jnp.float32)
    for k in range(CHUNKS):
        t16 = tgt_v[pl.ds(k * L, L)]
        v16 = vals_v[pl.ds(k * L, L)]
        acc = acc + jnp.where(t16 != PAD, v16, 0.0)
    acc_v[...] = acc
    pltpu.sync_copy(acc_v, out_hbm.at[pl.ds(wid * L, L)])


def _tc_body(x_ref, t_ref, s_ref, n_ref, acc_s, acc_n):
    i = pl.program_id(0)

    @pl.when(i == 0)
    def _init():
        acc_s[0] = 0.0
        acc_n[0] = 0.0

    xb = x_ref[...]
    tb = t_ref[0, 0, :]
    maskf = (tb != PAD).astype(jnp.float32)[:, None]
    acc_s[0] += jnp.sum(xb * maskf)
    acc_n[0] += jnp.sum(maskf)

    @pl.when(i == NB - 1)
    def _fin():
        s_ref[0, 0] = acc_s[0]
        n_ref[0, 0] = acc_n[0]


def _combine_body(g_ref, s_ref, n_ref, out_ref):
    tok = n_ref[0, 0]
    g = jnp.sum(g_ref[...])
    num = KCONST * tok - EPS * s_ref[0, 0] + (EPS - CONF) * g
    out_ref[0, 0] = num / tok


@jax.jit
def _tc_stream(x, t3):
    return pl.pallas_call(
        _tc_body,
        grid=(NB,),
        in_specs=[
            pl.BlockSpec((R, VOCAB), lambda i: (i, 0)),
            pl.BlockSpec((1, 1, R), lambda i: (i, 0, 0)),
        ],
        out_specs=[
            pl.BlockSpec(memory_space=pltpu.SMEM),
            pl.BlockSpec(memory_space=pltpu.SMEM),
        ],
        out_shape=[
            jax.ShapeDtypeStruct((1, 1), jnp.float32),
            jax.ShapeDtypeStruct((1, 1), jnp.float32),
        ],
        scratch_shapes=[
            pltpu.SMEM((1,), jnp.float32),
            pltpu.SMEM((1,), jnp.float32),
        ],
    )(x, t3)


@jax.jit
def _combine(g2, s, n):
    return pl.pallas_call(
        _combine_body,
        in_specs=[
            pl.BlockSpec((1, NW * L), lambda: (0, 0)),
            pl.BlockSpec(memory_space=pltpu.SMEM),
            pl.BlockSpec(memory_space=pltpu.SMEM),
        ],
        out_specs=pl.BlockSpec(memory_space=pltpu.SMEM),
        out_shape=jax.ShapeDtypeStruct((1, 1), jnp.float32),
    )(g2, s, n)


def _tcsg_body(x_ref, t_ref, s_ref, n_ref, g_ref, acc_s, acc_n, acc_g):
    i = pl.program_id(0)

    @pl.when(i == 0)
    def _init():
        acc_s[0] = 0.0
        acc_n[0] = 0.0
        acc_g[0] = 0.0

    xb = x_ref[...]
    tb = t_ref[0, 0, :]
    maskf = (tb != PAD).astype(jnp.float32)[:, None]
    xm = xb * maskf
    cols = lax.broadcasted_iota(jnp.int32, (R, VOCAB), 1)
    sel = (cols == tb[:, None]).astype(jnp.float32)
    acc_s[0] += jnp.sum(xm)
    acc_n[0] += jnp.sum(maskf)
    acc_g[0] += jnp.sum(xm * sel)

    @pl.when(i == NB - 1)
    def _fin():
        s_ref[0, 0] = acc_s[0]
        n_ref[0, 0] = acc_n[0]
        g_ref[0, 0] = acc_g[0]


@jax.jit
def _tc_stream_g(x, t3):
    return pl.pallas_call(
        _tcsg_body,
        grid=(NB,),
        in_specs=[
            pl.BlockSpec((R, VOCAB), lambda i: (i, 0)),
            pl.BlockSpec((1, 1, R), lambda i: (i, 0, 0)),
        ],
        out_specs=[
            pl.BlockSpec(memory_space=pltpu.SMEM),
            pl.BlockSpec(memory_space=pltpu.SMEM),
            pl.BlockSpec(memory_space=pltpu.SMEM),
        ],
        out_shape=[
            jax.ShapeDtypeStruct((1, 1), jnp.float32),
            jax.ShapeDtypeStruct((1, 1), jnp.float32),
            jax.ShapeDtypeStruct((1, 1), jnp.float32),
        ],
        scratch_shapes=[
            pltpu.SMEM((1,), jnp.float32),
            pltpu.SMEM((1,), jnp.float32),
            pltpu.SMEM((1,), jnp.float32),
        ],
    )(x, t3)


def kernel(x, target):
    t = target.astype(jnp.int32)
    s, n = _tc_stream(x, t.reshape(NB, 1, R))
    g = _sc_gather(x.reshape(ROWS * VOCAB), t)
    return _combine(g.reshape(1, NW * L), s, n)[0, 0]


# speedup vs baseline: 1.0680x; 1.0613x over previous
import functools
import math

import jax
import jax.numpy as jnp
from jax import lax
from jax.experimental import pallas as pl
from jax.experimental.pallas import tpu as pltpu
from jax.experimental.pallas import tpu_sc as plsc

VOCAB = 32000
PAD = 0
SMOOTH = 0.1
CONF = 1.0 - SMOOTH
EPS = SMOOTH / (VOCAB - 1)
KCONST = (VOCAB - 1) * EPS * math.log(EPS) + CONF * math.log(CONF)

ROWS = 2048
LANES = 128
XROWS = ROWS * (VOCAB // LANES)

NC, NS, L = 1, 16, 16
NW = NC * NS
BPW = ROWS // NW
CHUNKS = BPW // L

R = 64
NB = ROWS // R


@functools.partial(
    pl.kernel,
    out_type=jax.ShapeDtypeStruct((NW * L,), jnp.float32),
    mesh=plsc.VectorSubcoreMesh(
        core_axis_name="c", subcore_axis_name="s", num_cores=NC
    ),
    scratch_types=[
        pltpu.VMEM((BPW,), jnp.int32),
        pltpu.VMEM((BPW,), jnp.int32),
        pltpu.VMEM((BPW,), jnp.float32),
        pltpu.VMEM((L,), jnp.float32),
        pltpu.SemaphoreType.DMA,
    ],
)
def _sc_gather(xflat_hbm, tgt_hbm, out_hbm, tgt_v, idx_v, vals_v, acc_v, sem):
    wid = lax.axis_index("s") * NC + lax.axis_index("c")
    base = wid * BPW
    pltpu.sync_copy(tgt_hbm.at[pl.ds(base, BPW)], tgt_v)
    for k in range(CHUNKS):
        t16 = tgt_v[pl.ds(k * L, L)]
        row16 = (lax.iota(jnp.int32, L) + (base + k * L)) * VOCAB + t16
        idx_v[pl.ds(k * L, L)] = row16
    pltpu.async_copy(xflat_hbm.at[idx_v], vals_v, sem).wait()
    acc = jnp.zeros((L,), jnp.float32)
    for k in range(CHUNKS):
        t16 = tgt_v[pl.ds(k * L, L)]
        v16 = vals_v[pl.ds(k * L, L)]
        acc = acc + jnp.where(t16 != PAD, v16, 0.0)
    acc_v[...] = acc
    pltpu.sync_copy(acc_v, out_hbm.at[pl.ds(wid * L, L)])


def _tc_body(x_ref, t_ref, s_ref, n_ref, acc_s, acc_n):
    i = pl.program_id(0)

    @pl.when(i == 0)
    def _init():
        acc_s[0] = 0.0
        acc_n[0] = 0.0

    xb = x_ref[...]
    tb = t_ref[0, 0, :]
    maskf = (tb != PAD).astype(jnp.float32)
    rs = jnp.sum(xb, axis=1)
    acc_s[0] += jnp.sum(rs * maskf)
    acc_n[0] += jnp.sum(maskf)

    @pl.when(i == NB - 1)
    def _fin():
        s_ref[0, 0] = acc_s[0]
        n_ref[0, 0] = acc_n[0]


def _combine_body(g_ref, s_ref, n_ref, out_ref):
    tok = n_ref[0, 0]
    g = jnp.sum(g_ref[...])
    num = KCONST * tok - EPS * s_ref[0, 0] + (EPS - CONF) * g
    out_ref[0, 0] = num / tok


@jax.jit
def _tc_stream(x, t3):
    return pl.pallas_call(
        _tc_body,
        grid=(NB,),
        in_specs=[
            pl.BlockSpec((R, VOCAB), lambda i: (i, 0)),
            pl.BlockSpec((1, 1, R), lambda i: (i, 0, 0)),
        ],
        out_specs=[
            pl.BlockSpec(memory_space=pltpu.SMEM),
            pl.BlockSpec(memory_space=pltpu.SMEM),
        ],
        out_shape=[
            jax.ShapeDtypeStruct((1, 1), jnp.float32),
            jax.ShapeDtypeStruct((1, 1), jnp.float32),
        ],
        scratch_shapes=[
            pltpu.SMEM((1,), jnp.float32),
            pltpu.SMEM((1,), jnp.float32),
        ],
    )(x, t3)


@jax.jit
def _combine(g2, s, n):
    return pl.pallas_call(
        _combine_body,
        in_specs=[
            pl.BlockSpec((1, NW * L), lambda: (0, 0)),
            pl.BlockSpec(memory_space=pltpu.SMEM),
            pl.BlockSpec(memory_space=pltpu.SMEM),
        ],
        out_specs=pl.BlockSpec(memory_space=pltpu.SMEM),
        out_shape=jax.ShapeDtypeStruct((1, 1), jnp.float32),
    )(g2, s, n)


def _tcsg_body(x_ref, t_ref, s_ref, n_ref, g_ref, acc_s, acc_n, acc_g):
    i = pl.program_id(0)

    @pl.when(i == 0)
    def _init():
        acc_s[0] = 0.0
        acc_n[0] = 0.0
        acc_g[0] = 0.0

    xb = x_ref[...]
    tb = t_ref[0, 0, :]
    maskf = (tb != PAD).astype(jnp.float32)[:, None]
    xm = xb * maskf
    cols = lax.broadcasted_iota(jnp.int32, (R, VOCAB), 1)
    sel = (cols == tb[:, None]).astype(jnp.float32)
    acc_s[0] += jnp.sum(xm)
    acc_n[0] += jnp.sum(maskf)
    acc_g[0] += jnp.sum(xm * sel)

    @pl.when(i == NB - 1)
    def _fin():
        s_ref[0, 0] = acc_s[0]
        n_ref[0, 0] = acc_n[0]
        g_ref[0, 0] = acc_g[0]


@jax.jit
def _tc_stream_g(x, t3):
    return pl.pallas_call(
        _tcsg_body,
        grid=(NB,),
        in_specs=[
            pl.BlockSpec((R, VOCAB), lambda i: (i, 0)),
            pl.BlockSpec((1, 1, R), lambda i: (i, 0, 0)),
        ],
        out_specs=[
            pl.BlockSpec(memory_space=pltpu.SMEM),
            pl.BlockSpec(memory_space=pltpu.SMEM),
            pl.BlockSpec(memory_space=pltpu.SMEM),
        ],
        out_shape=[
            jax.ShapeDtypeStruct((1, 1), jnp.float32),
            jax.ShapeDtypeStruct((1, 1), jnp.float32),
            jax.ShapeDtypeStruct((1, 1), jnp.float32),
        ],
        scratch_shapes=[
            pltpu.SMEM((1,), jnp.float32),
            pltpu.SMEM((1,), jnp.float32),
            pltpu.SMEM((1,), jnp.float32),
        ],
    )(x, t3)


def kernel(x, target):
    t = target.astype(jnp.int32)
    s, n = _tc_stream(x, t.reshape(NB, 1, R))
    g = _sc_gather(x.reshape(ROWS * VOCAB), t)
    return _combine(g.reshape(1, NW * L), s, n)[0, 0]


# speedup vs baseline: 1.0720x; 1.0037x over previous
import functools
import math

import jax
import jax.numpy as jnp
from jax import lax
from jax.experimental import pallas as pl
from jax.experimental.pallas import tpu as pltpu
from jax.experimental.pallas import tpu_sc as plsc

VOCAB = 32000
PAD = 0
SMOOTH = 0.1
CONF = 1.0 - SMOOTH
EPS = SMOOTH / (VOCAB - 1)
KCONST = (VOCAB - 1) * EPS * math.log(EPS) + CONF * math.log(CONF)

ROWS = 2048
LANES = 128
XROWS = ROWS * (VOCAB // LANES)

NC, NS, L = 1, 16, 16
NW = NC * NS
BPW = ROWS // NW
CHUNKS = BPW // L

R = 128
NB = ROWS // R


@functools.partial(
    pl.kernel,
    out_type=jax.ShapeDtypeStruct((NW * L,), jnp.float32),
    mesh=plsc.VectorSubcoreMesh(
        core_axis_name="c", subcore_axis_name="s", num_cores=NC
    ),
    scratch_types=[
        pltpu.VMEM((BPW,), jnp.int32),
        pltpu.VMEM((BPW,), jnp.int32),
        pltpu.VMEM((BPW,), jnp.float32),
        pltpu.VMEM((L,), jnp.float32),
        pltpu.SemaphoreType.DMA,
    ],
)
def _sc_gather(xflat_hbm, tgt_hbm, out_hbm, tgt_v, idx_v, vals_v, acc_v, sem):
    wid = lax.axis_index("s") * NC + lax.axis_index("c")
    base = wid * BPW
    pltpu.sync_copy(tgt_hbm.at[pl.ds(base, BPW)], tgt_v)
    for k in range(CHUNKS):
        t16 = tgt_v[pl.ds(k * L, L)]
        row16 = (lax.iota(jnp.int32, L) + (base + k * L)) * VOCAB + t16
        idx_v[pl.ds(k * L, L)] = row16
    pltpu.async_copy(xflat_hbm.at[idx_v], vals_v, sem).wait()
    acc = jnp.zeros((L,), jnp.float32)
    for k in range(CHUNKS):
        t16 = tgt_v[pl.ds(k * L, L)]
        v16 = vals_v[pl.ds(k * L, L)]
        acc = acc + jnp.where(t16 != PAD, v16, 0.0)
    acc_v[...] = acc
    pltpu.sync_copy(acc_v, out_hbm.at[pl.ds(wid * L, L)])


def _tc_body(x_ref, t_ref, s_ref, n_ref, acc_s, acc_n):
    i = pl.program_id(0)

    @pl.when(i == 0)
    def _init():
        acc_s[0] = 0.0
        acc_n[0] = 0.0

    xb = x_ref[...]
    tb = t_ref[0, 0, :]
    maskf = (tb != PAD).astype(jnp.float32)
    rs = jnp.sum(xb, axis=1)
    acc_s[0] += jnp.sum(rs * maskf)
    acc_n[0] += jnp.sum(maskf)

    @pl.when(i == NB - 1)
    def _fin():
        s_ref[0, 0] = acc_s[0]
        n_ref[0, 0] = acc_n[0]


def _combine_body(g_ref, s_ref, n_ref, out_ref):
    tok = n_ref[0, 0]
    g = jnp.sum(g_ref[...])
    num = KCONST * tok - EPS * s_ref[0, 0] + (EPS - CONF) * g
    out_ref[0, 0] = num / tok


@jax.jit
def _tc_stream(x, t3):
    return pl.pallas_call(
        _tc_body,
        grid=(NB,),
        in_specs=[
            pl.BlockSpec((R, VOCAB), lambda i: (i, 0)),
            pl.BlockSpec((1, 1, R), lambda i: (i, 0, 0)),
        ],
        out_specs=[
            pl.BlockSpec(memory_space=pltpu.SMEM),
            pl.BlockSpec(memory_space=pltpu.SMEM),
        ],
        out_shape=[
            jax.ShapeDtypeStruct((1, 1), jnp.float32),
            jax.ShapeDtypeStruct((1, 1), jnp.float32),
        ],
        scratch_shapes=[
            pltpu.SMEM((1,), jnp.float32),
            pltpu.SMEM((1,), jnp.float32),
        ],
    )(x, t3)


@jax.jit
def _combine(g2, s, n):
    return pl.pallas_call(
        _combine_body,
        in_specs=[
            pl.BlockSpec((1, NW * L), lambda: (0, 0)),
            pl.BlockSpec(memory_space=pltpu.SMEM),
            pl.BlockSpec(memory_space=pltpu.SMEM),
        ],
        out_specs=pl.BlockSpec(memory_space=pltpu.SMEM),
        out_shape=jax.ShapeDtypeStruct((1, 1), jnp.float32),
    )(g2, s, n)


def _tcsg_body(x_ref, t_ref, s_ref, n_ref, g_ref, acc_s, acc_n, acc_g):
    i = pl.program_id(0)

    @pl.when(i == 0)
    def _init():
        acc_s[0] = 0.0
        acc_n[0] = 0.0
        acc_g[0] = 0.0

    xb = x_ref[...]
    tb = t_ref[0, 0, :]
    maskf = (tb != PAD).astype(jnp.float32)[:, None]
    xm = xb * maskf
    cols = lax.broadcasted_iota(jnp.int32, (R, VOCAB), 1)
    sel = (cols == tb[:, None]).astype(jnp.float32)
    acc_s[0] += jnp.sum(xm)
    acc_n[0] += jnp.sum(maskf)
    acc_g[0] += jnp.sum(xm * sel)

    @pl.when(i == NB - 1)
    def _fin():
        s_ref[0, 0] = acc_s[0]
        n_ref[0, 0] = acc_n[0]
        g_ref[0, 0] = acc_g[0]


@jax.jit
def _tc_stream_g(x, t3):
    return pl.pallas_call(
        _tcsg_body,
        grid=(NB,),
        in_specs=[
            pl.BlockSpec((R, VOCAB), lambda i: (i, 0)),
            pl.BlockSpec((1, 1, R), lambda i: (i, 0, 0)),
        ],
        out_specs=[
            pl.BlockSpec(memory_space=pltpu.SMEM),
            pl.BlockSpec(memory_space=pltpu.SMEM),
            pl.BlockSpec(memory_space=pltpu.SMEM),
        ],
        out_shape=[
            jax.ShapeDtypeStruct((1, 1), jnp.float32),
            jax.ShapeDtypeStruct((1, 1), jnp.float32),
            jax.ShapeDtypeStruct((1, 1), jnp.float32),
        ],
        scratch_shapes=[
            pltpu.SMEM((1,), jnp.float32),
            pltpu.SMEM((1,), jnp.float32),
            pltpu.SMEM((1,), jnp.float32),
        ],
    )(x, t3)


def kernel(x, target):
    t = target.astype(jnp.int32)
    s, n = _tc_stream(x, t.reshape(NB, 1, R))
    g = _sc_gather(x.reshape(ROWS * VOCAB), t)
    return _combine(g.reshape(1, NW * L), s, n)[0, 0]


# speedup vs baseline: 2.9549x; 2.7565x over previous
import functools
import math

import jax
import jax.numpy as jnp
from jax import lax
from jax.experimental import pallas as pl
from jax.experimental.pallas import tpu as pltpu
from jax.experimental.pallas import tpu_sc as plsc

VOCAB = 32000
PAD = 0
SMOOTH = 0.1
CONF = 1.0 - SMOOTH
EPS = SMOOTH / (VOCAB - 1)
KCONST = (VOCAB - 1) * EPS * math.log(EPS) + CONF * math.log(CONF)

ROWS = 2048

NS, L = 16, 16
NW = NS
BPW = ROWS // NW
CHUNKS = BPW // L

R = 128
NB = ROWS // R


def _tc_body(x_ref, t_ref, rs_ref, g_ref):
    xb = x_ref[...]
    tb = t_ref[0, 0, :]
    rs_ref[0, 0, :] = jnp.sum(xb, axis=1)
    cols = lax.broadcasted_iota(jnp.int32, (R, VOCAB), 1)
    sel = cols == tb[:, None]
    g_ref[0, 0, :] = jnp.sum(jnp.where(sel, xb, 0.0), axis=1)


@jax.jit
def _tc_stream(x, t3):
    return pl.pallas_call(
        _tc_body,
        grid=(NB,),
        in_specs=[
            pl.BlockSpec((R, VOCAB), lambda i: (i, 0)),
            pl.BlockSpec((1, 1, R), lambda i: (i, 0, 0)),
        ],
        out_specs=[
            pl.BlockSpec((1, 1, R), lambda i: (i, 0, 0)),
            pl.BlockSpec((1, 1, R), lambda i: (i, 0, 0)),
        ],
        out_shape=[
            jax.ShapeDtypeStruct((NB, 1, R), jnp.float32),
            jax.ShapeDtypeStruct((NB, 1, R), jnp.float32),
        ],
    )(x, t3)


@functools.partial(
    pl.kernel,
    out_type=jax.ShapeDtypeStruct((2, L), jnp.float32),
    mesh=plsc.VectorSubcoreMesh(
        core_axis_name="c", subcore_axis_name="s", num_cores=1
    ),
    scratch_types=[
        pltpu.VMEM((BPW,), jnp.float32),
        pltpu.VMEM((BPW,), jnp.float32),
        pltpu.VMEM((BPW,), jnp.int32),
        pltpu.VMEM((2, L), jnp.float32),
        pltpu.VMEM((NW, 2, L), jnp.float32),
        pltpu.VMEM((2, L), jnp.float32),
        pltpu.VMEM_SHARED((NW, 2, L), jnp.float32),
    ],
)
def _sc_reduce(rs_hbm, g_hbm, t_hbm, out_hbm, rs_v, g_v, t_v, part_v, all_v,
               out_v, shared):
    wid = lax.axis_index("s")
    base = wid * BPW
    pltpu.sync_copy(rs_hbm.at[pl.ds(base, BPW)], rs_v)
    pltpu.sync_copy(g_hbm.at[pl.ds(base, BPW)], g_v)
    pltpu.sync_copy(t_hbm.at[pl.ds(base, BPW)], t_v)
    acc_n = jnp.zeros((L,), jnp.float32)
    acc_t = jnp.zeros((L,), jnp.float32)
    for k in range(CHUNKS):
        t16 = t_v[pl.ds(k * L, L)]
        rs16 = rs_v[pl.ds(k * L, L)]
        g16 = g_v[pl.ds(k * L, L)]
        mask = t16 != PAD
        contrib = (EPS - CONF) * g16 - EPS * rs16
        acc_n = acc_n + jnp.where(mask, contrib, 0.0)
        acc_t = acc_t + jnp.where(mask, 1.0, 0.0)
    part_v[0, :] = acc_n
    part_v[1, :] = acc_t
    pltpu.sync_copy(part_v, shared.at[wid])
    plsc.subcore_barrier()

    @pl.when(wid == 0)
    def _final():
        pltpu.sync_copy(shared, all_v)
        num = jnp.zeros((L,), jnp.float32)
        tokv = jnp.zeros((L,), jnp.float32)
        for w in range(NW):
            num = num + all_v[w, 0]
            tokv = tokv + all_v[w, 1]
        out_v[0, :] = num
        out_v[1, :] = tokv
        pltpu.sync_copy(out_v, out_hbm)


def _combine_body(p_ref, out_ref):
    num = jnp.sum(p_ref[0, :])
    tok = jnp.sum(p_ref[1, :])
    out_ref[0, 0] = (KCONST * tok + num) / tok


@jax.jit
def _combine(p):
    return pl.pallas_call(
        _combine_body,
        in_specs=[pl.BlockSpec((2, L), lambda: (0, 0))],
        out_specs=pl.BlockSpec(memory_space=pltpu.SMEM),
        out_shape=jax.ShapeDtypeStruct((1, 1), jnp.float32),
    )(p)


def kernel(x, target):
    t = target.astype(jnp.int32)
    rs3, g3 = _tc_stream(x, t.reshape(NB, 1, R))
    p = _sc_reduce(rs3.reshape(ROWS), g3.reshape(ROWS), t)
    return _combine(p)[0, 0]


# speedup vs baseline: 2.9821x; 1.0092x over previous
import functools
import math

import jax
import jax.numpy as jnp
from jax import lax
from jax.experimental import pallas as pl
from jax.experimental.pallas import tpu as pltpu
from jax.experimental.pallas import tpu_sc as plsc

VOCAB = 32000
PAD = 0
SMOOTH = 0.1
CONF = 1.0 - SMOOTH
EPS = SMOOTH / (VOCAB - 1)
KCONST = (VOCAB - 1) * EPS * math.log(EPS) + CONF * math.log(CONF)

ROWS = 2048

NS, L = 16, 16
NW = NS
BPW = ROWS // NW
CHUNKS = BPW // L

R = 128
NB = ROWS // R


def _tc_body(x_ref, t_ref, rs_ref, g_ref):
    xb = x_ref[...]
    tb = t_ref[0, 0, :]
    rs_ref[0, 0, :] = jnp.sum(xb, axis=1)
    cols = lax.broadcasted_iota(jnp.int32, (R, VOCAB), 1)
    sel = cols == tb[:, None]
    g_ref[0, 0, :] = jnp.sum(jnp.where(sel, xb, 0.0), axis=1)


@jax.jit
def _tc_stream(x, t3):
    return pl.pallas_call(
        _tc_body,
        grid=(NB,),
        in_specs=[
            pl.BlockSpec((R, VOCAB), lambda i: (i, 0)),
            pl.BlockSpec((1, 1, R), lambda i: (i, 0, 0)),
        ],
        out_specs=[
            pl.BlockSpec((1, 1, R), lambda i: (i, 0, 0)),
            pl.BlockSpec((1, 1, R), lambda i: (i, 0, 0)),
        ],
        out_shape=[
            jax.ShapeDtypeStruct((NB, 1, R), jnp.float32),
            jax.ShapeDtypeStruct((NB, 1, R), jnp.float32),
        ],
    )(x, t3)


@functools.partial(
    pl.kernel,
    out_type=jax.ShapeDtypeStruct((NW, 2, L), jnp.float32),
    mesh=plsc.VectorSubcoreMesh(
        core_axis_name="c", subcore_axis_name="s", num_cores=1
    ),
    scratch_types=[
        pltpu.VMEM((BPW,), jnp.float32),
        pltpu.VMEM((BPW,), jnp.float32),
        pltpu.VMEM((BPW,), jnp.int32),
        pltpu.VMEM((2, L), jnp.float32),
    ],
)
def _sc_reduce(rs_hbm, g_hbm, t_hbm, out_hbm, rs_v, g_v, t_v, part_v):
    wid = lax.axis_index("s")
    base = wid * BPW
    pltpu.sync_copy(rs_hbm.at[pl.ds(base, BPW)], rs_v)
    pltpu.sync_copy(g_hbm.at[pl.ds(base, BPW)], g_v)
    pltpu.sync_copy(t_hbm.at[pl.ds(base, BPW)], t_v)
    acc_n = jnp.zeros((L,), jnp.float32)
    acc_t = jnp.zeros((L,), jnp.float32)
    for k in range(CHUNKS):
        t16 = t_v[pl.ds(k * L, L)]
        rs16 = rs_v[pl.ds(k * L, L)]
        g16 = g_v[pl.ds(k * L, L)]
        mask = t16 != PAD
        contrib = (EPS - CONF) * g16 - EPS * rs16
        acc_n = acc_n + jnp.where(mask, contrib, 0.0)
        acc_t = acc_t + jnp.where(mask, 1.0, 0.0)
    part_v[0, :] = acc_n
    part_v[1, :] = acc_t
    pltpu.sync_copy(part_v, out_hbm.at[wid])


def _combine_body(p_ref, out_ref):
    num = jnp.sum(p_ref[:, 0, :])
    tok = jnp.sum(p_ref[:, 1, :])
    out_ref[0, 0] = (KCONST * tok + num) / tok


@jax.jit
def _combine(p):
    return pl.pallas_call(
        _combine_body,
        in_specs=[pl.BlockSpec((NW, 2, L), lambda: (0, 0, 0))],
        out_specs=pl.BlockSpec(memory_space=pltpu.SMEM),
        out_shape=jax.ShapeDtypeStruct((1, 1), jnp.float32),
    )(p)


def kernel(x, target):
    t = target.astype(jnp.int32)
    rs3, g3 = _tc_stream(x, t.reshape(NB, 1, R))
    p = _sc_reduce(rs3.reshape(ROWS), g3.reshape(ROWS), t)
    return _combine(p)[0, 0]
